# baseline (speedup 1.0000x reference)
import functools
import jax
import jax.numpy as jnp
from jax import lax
from jax.experimental import pallas as pl
from jax.experimental.pallas import tpu as pltpu
from jax.experimental.pallas import tpu_sc as plsc

_RS = 32
_NB = 3
_D = 768
_L = 16


def _sc_add(in_flat, pos_table):
    n_rows, d = in_flat.shape
    seq = pos_table.shape[0]
    batch = n_rows // seq
    n_workers = 32
    seq_per_w = seq // n_workers
    n_steps = seq_per_w // _RS
    n_chunks = n_steps * batch
    mesh = plsc.VectorSubcoreMesh(core_axis_name="c", subcore_axis_name="s")

    @functools.partial(
        pl.kernel,
        mesh=mesh,
        out_type=jax.ShapeDtypeStruct((n_rows, d), jnp.float32),
        scratch_types=[
            pltpu.VMEM((_NB, _RS, _D), jnp.float32),
            pltpu.VMEM((2, _RS, _D), jnp.float32),
            pltpu.SemaphoreType.DMA((_NB,)),
            pltpu.SemaphoreType.DMA((_NB,)),
            pltpu.SemaphoreType.DMA((2,)),
        ],
    )
    def k(in_hbm, pos_hbm, out_hbm, ibuf, posv, isems, osems, psems):
        wid = lax.axis_index("s") * 2 + lax.axis_index("c")
        seq0 = wid * seq_per_w

        def row0_of(c):
            return (c % batch) * seq + seq0 + (c // batch) * _RS

        def in_cp(c):
            return pltpu.make_async_copy(
                in_hbm.at[pl.ds(row0_of(c), _RS)],
                ibuf.at[lax.rem(c, _NB)],
                isems.at[lax.rem(c, _NB)],
            )

        def out_cp(c):
            return pltpu.make_async_copy(
                ibuf.at[lax.rem(c, _NB)],
                out_hbm.at[pl.ds(row0_of(c), _RS)],
                osems.at[lax.rem(c, _NB)],
            )

        def pos_cp(t):
            return pltpu.make_async_copy(
                pos_hbm.at[pl.ds(seq0 + t * _RS, _RS)],
                posv.at[lax.rem(t, 2)],
                psems.at[lax.rem(t, 2)],
            )

        pos_cp(0).start()
        pos_cp(1).start()
        for c0 in range(_NB):
            in_cp(c0).start()

        def chunk(c, _):
            t = c // batch
            b = c % batch
            prm = lax.rem(t, 2)

            @pl.when(c >= 1)
            def _():
                out_cp(c - 1).wait()
                @pl.when(c - 1 + _NB < n_chunks)
                def _():
                    in_cp(c - 1 + _NB).start()

            in_cp(c).wait()

            @pl.when(b == 0)
            def _():
                pos_cp(t).wait()

            slot = lax.rem(c, _NB)

            def add_row(r, _):
                for cc in range(_D // _L):
                    sl = pl.ds(cc * _L, _L)
                    plsc.addupdate(ibuf.at[slot, r, sl], posv[prm, r, sl])
                return ()

            lax.fori_loop(0, _RS, add_row, ())
            out_cp(c).start()

            @pl.when(jnp.logical_and(b == batch - 1, t + 2 < n_steps))
            def _():
                pos_cp(t + 2).start()

            return ()

        lax.fori_loop(0, n_chunks, chunk, ())
        out_cp(n_chunks - 1).wait()

    return k(in_flat, pos_table)


def kernel(input_embeds, pos_table):
    b, s, d = input_embeds.shape
    out = _sc_add(input_embeds.reshape(b * s, d), pos_table)
    return out.reshape(b, s, d)


# speedup vs baseline: 3.2205x; 3.2205x over previous
import jax
import jax.numpy as jnp
from jax.experimental import pallas as pl
from jax.experimental.pallas import tpu as pltpu

_C = 1024
_NBUF = 4
_D = 768


def _body(in_hbm, pos_hbm, out_hbm, ibuf, obuf, posv, isems, osems, psems):
    n_rows = in_hbm.shape[0]
    seq = pos_hbm.shape[0]
    n_chunks = n_rows // _C
    pos_chunks = seq // _C

    def in_copy(c):
        return pltpu.make_async_copy(
            in_hbm.at[pl.ds(c * _C, _C)], ibuf.at[c % _NBUF], isems.at[c % _NBUF]
        )

    def out_copy(c):
        return pltpu.make_async_copy(
            obuf.at[c % _NBUF], out_hbm.at[pl.ds(c * _C, _C)], osems.at[c % _NBUF]
        )

    def pos_copy(p):
        return pltpu.make_async_copy(
            pos_hbm.at[pl.ds(p * _C, _C)], posv.at[pl.ds(p * _C, _C)], psems.at[p]
        )

    pos_copy(0).start()
    for k in range(_NBUF):
        in_copy(k).start()
        if k + 1 < pos_chunks:
            pos_copy(k + 1).start()
    for p in range(_NBUF + 1, pos_chunks):
        pos_copy(p).start()

    for c in range(n_chunks):
        slot = c % _NBUF
        if c >= _NBUF:
            out_copy(c - _NBUF).wait()
        in_copy(c).wait()
        if c < pos_chunks:
            pos_copy(c).wait()
        obuf[slot] = ibuf[slot] + posv[pl.ds((c * _C) % seq, _C)]
        out_copy(c).start()
        if c + _NBUF < n_chunks:
            in_copy(c + _NBUF).start()

    for c in range(n_chunks - _NBUF, n_chunks):
        out_copy(c).wait()


def kernel(input_embeds, pos_table):
    batch, seq, d = input_embeds.shape
    flat = input_embeds.reshape(batch * seq, d)

    out = pl.pallas_call(
        _body,
        in_specs=[
            pl.BlockSpec(memory_space=pl.ANY),
            pl.BlockSpec(memory_space=pl.ANY),
        ],
        out_specs=pl.BlockSpec(memory_space=pl.ANY),
        out_shape=jax.ShapeDtypeStruct((batch * seq, d), input_embeds.dtype),
        scratch_shapes=[
            pltpu.VMEM((_NBUF, _C, _D), jnp.float32),
            pltpu.VMEM((_NBUF, _C, _D), jnp.float32),
            pltpu.VMEM((8192, _D), jnp.float32),
            pltpu.SemaphoreType.DMA((_NBUF,)),
            pltpu.SemaphoreType.DMA((_NBUF,)),
            pltpu.SemaphoreType.DMA((8192 // _C,)),
        ],
    )(flat, pos_table)
    return out.reshape(batch, seq, d)
